# baseline (speedup 1.0000x reference)
import functools

import jax
import jax.numpy as jnp
from jax import lax
from jax.experimental import pallas as pl
from jax.experimental.pallas import tpu as pltpu
from jax.experimental.pallas import tpu_sc as plsc

MAXN = 200000
NEWB = 4096
RES = MAXN - NEWB
KEY = 20000

NC = 1
NS = 16
NW = NC * NS
BSUB = 128
SUBG = 10
CH = SUBG * BSUB
PAD = NW * CH
TRASH = 2048
GRP = BSUB // 16
BBLK = 64
NBLK = CH // BBLK


def _sc_sample(feat, new_feat, reserved, samp1d):
    mesh = plsc.VectorSubcoreMesh(
        core_axis_name="c", subcore_axis_name="s", num_cores=NC)

    @functools.partial(
        pl.kernel,
        mesh=mesh,
        out_type=jax.ShapeDtypeStruct((PAD + TRASH, 256), jnp.float32),
        scratch_types=[
            pltpu.VMEM((CH,), jnp.int32),
            pltpu.VMEM((SUBG, BSUB), jnp.int32),
            pltpu.VMEM((SUBG, BSUB), jnp.int32),
            pltpu.VMEM((SUBG, BSUB), jnp.int32),
            pltpu.VMEM((CH + 16,), jnp.int32),
            pltpu.VMEM((CH + 16,), jnp.int32),
            pltpu.VMEM((NBLK, BBLK), jnp.int32),
            pltpu.VMEM((NBLK, BBLK), jnp.int32),
            pltpu.VMEM((32,), jnp.int32),
            pltpu.VMEM((BSUB, 256), jnp.float32),
            pltpu.VMEM((BSUB, 256), jnp.float32),
            pltpu.VMEM((BBLK, 256), jnp.float32),
            pltpu.SemaphoreType.DMA,
            pltpu.SemaphoreType.DMA,
            pltpu.SemaphoreType.DMA,
            pltpu.SemaphoreType.DMA,
        ],
    )
    def k(feat_h, new_h, res_h, samp_h, out_h,
          s_b, sc_b, r_b, ia_b, ibf, pbf, ibg, pbg, z_b,
          buf_a0, buf_a1, buf_b,
          sem_r, sem_ga, sem_w, sem_b):
        wid = lax.axis_index("s") * NC + lax.axis_index("c")
        base = wid * CH
        iota = lax.iota(jnp.int32, 16)
        pltpu.sync_copy(samp_h.at[pl.ds(wid * CH, CH)], s_b)
        z_b[pl.ds(0, 16)] = jnp.zeros((16,), jnp.int32)
        r_copies = []
        for g in range(SUBG):
            for t in range(GRP):
                j = g * GRP + t
                s = s_b[pl.ds(j * 16, 16)]
                pos = base + j * 16 + iota
                sc_b[g, pl.ds(t * 16, 16)] = jnp.minimum(s, RES - 1)
                ibf[pl.ds(j * 16, 16)] = pos & (NEWB - 1)
                pbf[pl.ds(j * 16, 16)] = PAD + (pos & (TRASH - 1))
            r_copies.append(
                pltpu.async_copy(res_h.at[sc_b.at[g]], r_b.at[g], sem_r))
        bufs_a = [buf_a0, buf_a1]

        def a_gather(g):
            return pltpu.async_copy(
                feat_h.at[ia_b.at[g]], bufs_a[g % 2], sem_ga)

        ga = {}
        n_b = jnp.int32(0)
        for g in range(SUBG):
            r_copies[g].wait()
            for t in range(GRP):
                j = g * GRP + t
                s = s_b[pl.ds(j * 16, 16)]
                r = r_b[g, pl.ds(t * 16, 16)]
                pos = base + j * 16 + iota
                m = s < RES
                ia_b[g, pl.ds(t * 16, 16)] = jnp.where(m, r, pos)
                mb = s >= RES
                cum = jnp.where(mb, 1, 0)
                for sh in (1, 2, 4, 8):
                    z_b[pl.ds(16, 16)] = cum
                    cum = cum + z_b[pl.ds(16 - sh, 16)]
                cnt = cum[15]

                @pl.when(cnt > 0)
                def _():
                    cntv = jnp.full((16,), cnt, jnp.int32)
                    sel = jnp.zeros((16,), jnp.int32)
                    for l in range(16):
                        sel = sel + jnp.where(cum[l] <= iota, 1, 0)
                    sel = jnp.minimum(sel, 15)
                    pk_i = (s - RES).at[sel].get(mode="promise_in_bounds")
                    pk_p = pos.at[sel].get(mode="promise_in_bounds")
                    pk_i = jnp.where(iota < cntv, pk_i, pos & (NEWB - 1))
                    pk_p = jnp.where(iota < cntv, pk_p,
                                     PAD + (pos & (TRASH - 1)))
                    ibf[pl.ds(n_b, 16)] = pk_i
                    pbf[pl.ds(n_b, 16)] = pk_p
                n_b = n_b + cnt
            if g < 2:
                ga[g] = a_gather(g)
        for q in range(NBLK):
            for t in range(BBLK // 16):
                j = q * (BBLK // 16) + t
                ibg[q, pl.ds(t * 16, 16)] = ibf[pl.ds(j * 16, 16)]
                pbg[q, pl.ds(t * 16, 16)] = pbf[pl.ds(j * 16, 16)]
        gb0 = pltpu.async_copy(new_h.at[ibg.at[0]], buf_b, sem_b)
        wa = {}
        for g in range(SUBG):
            ga[g].wait()
            wa[g] = pltpu.async_copy(
                bufs_a[g % 2], out_h.at[pl.ds(base + g * BSUB, BSUB)], sem_w)
            if g + 2 < SUBG:
                wa[g].wait()
                ga[g + 2] = a_gather(g + 2)
        for g in range(max(0, SUBG - 2), SUBG):
            wa[g].wait()
        gb0.wait()

        @pl.when(n_b > 0)
        def _():
            pltpu.async_copy(buf_b, out_h.at[pbg.at[0]], sem_b).wait()
        for q in range(1, NBLK):
            @pl.when(n_b > q * BBLK)
            def _():
                pltpu.async_copy(new_h.at[ibg.at[q]], buf_b, sem_b).wait()
                pltpu.async_copy(buf_b, out_h.at[pbg.at[q]], sem_b).wait()

    return k(feat, new_feat, reserved, samp1d)


def kernel(feat, new_feat, reserved_ind, sampled_ind):
    pad = jnp.zeros((PAD - KEY,), dtype=sampled_ind.dtype)
    samp1d = jnp.concatenate([sampled_ind, pad])
    out = _sc_sample(feat, new_feat, reserved_ind, samp1d)
    return out[:KEY]


# speedup vs baseline: 1.1582x; 1.1582x over previous
import functools

import jax
import jax.numpy as jnp
from jax import lax
from jax.experimental import pallas as pl
from jax.experimental.pallas import tpu as pltpu
from jax.experimental.pallas import tpu_sc as plsc

MAXN = 200000
NEWB = 4096
RES = MAXN - NEWB
KEY = 20000

NC = 2
NS = 16
NW = NC * NS
BSUB = 128
SUBG = 5
CH = SUBG * BSUB
PAD = NW * CH
TRASH = 2048
GRP = BSUB // 16
BBLK = 64
NBLK = CH // BBLK


def _sc_sample(feat, new_feat, reserved, samp1d):
    mesh = plsc.VectorSubcoreMesh(core_axis_name="c", subcore_axis_name="s")

    @functools.partial(
        pl.kernel,
        mesh=mesh,
        out_type=jax.ShapeDtypeStruct((PAD + TRASH, 256), jnp.float32),
        scratch_types=[
            pltpu.VMEM((CH,), jnp.int32),
            pltpu.VMEM((SUBG, BSUB), jnp.int32),
            pltpu.VMEM((SUBG, BSUB), jnp.int32),
            pltpu.VMEM((SUBG, BSUB), jnp.int32),
            pltpu.VMEM((CH + 16,), jnp.int32),
            pltpu.VMEM((CH + 16,), jnp.int32),
            pltpu.VMEM((NBLK, BBLK), jnp.int32),
            pltpu.VMEM((NBLK, BBLK), jnp.int32),
            pltpu.VMEM((32,), jnp.int32),
            pltpu.VMEM((BSUB, 256), jnp.float32),
            pltpu.VMEM((BSUB, 256), jnp.float32),
            pltpu.VMEM((BSUB, 256), jnp.float32),
            pltpu.VMEM((BBLK, 256), jnp.float32),
            pltpu.SemaphoreType.DMA,
            pltpu.SemaphoreType.DMA,
            pltpu.SemaphoreType.DMA,
            pltpu.SemaphoreType.DMA,
        ],
    )
    def k(feat_h, new_h, res_h, samp_h, out_h,
          s_b, sc_b, r_b, ia_b, ibf, pbf, ibg, pbg, z_b,
          buf_a0, buf_a1, buf_a2, buf_b,
          sem_r, sem_ga, sem_w, sem_b):
        wid = lax.axis_index("s") * NC + lax.axis_index("c")
        base = wid * CH
        iota = lax.iota(jnp.int32, 16)
        pltpu.sync_copy(samp_h.at[pl.ds(wid * CH, CH)], s_b)
        z_b[pl.ds(0, 16)] = jnp.zeros((16,), jnp.int32)
        r_copies = []
        for g in range(SUBG):
            for t in range(GRP):
                j = g * GRP + t
                s = s_b[pl.ds(j * 16, 16)]
                pos = base + j * 16 + iota
                sc_b[g, pl.ds(t * 16, 16)] = jnp.minimum(s, RES - 1)
                ibf[pl.ds(j * 16, 16)] = pos & (NEWB - 1)
                pbf[pl.ds(j * 16, 16)] = PAD + (pos & (TRASH - 1))
            r_copies.append(
                pltpu.async_copy(res_h.at[sc_b.at[g]], r_b.at[g], sem_r))
        bufs_a = [buf_a0, buf_a1, buf_a2]

        def a_gather(g):
            return pltpu.async_copy(
                feat_h.at[ia_b.at[g]], bufs_a[g % 3], sem_ga)

        ga = {}
        n_b = jnp.int32(0)
        for g in range(SUBG):
            r_copies[g].wait()
            for t in range(GRP):
                j = g * GRP + t
                s = s_b[pl.ds(j * 16, 16)]
                r = r_b[g, pl.ds(t * 16, 16)]
                pos = base + j * 16 + iota
                m = s < RES
                ia_b[g, pl.ds(t * 16, 16)] = jnp.where(m, r, pos)
                mb = s >= RES
                cum = jnp.where(mb, 1, 0)
                for sh in (1, 2, 4, 8):
                    z_b[pl.ds(16, 16)] = cum
                    cum = cum + z_b[pl.ds(16 - sh, 16)]
                cnt = cum[15]

                @pl.when(cnt > 0)
                def _():
                    cntv = jnp.full((16,), cnt, jnp.int32)
                    sel = jnp.zeros((16,), jnp.int32)
                    for l in range(16):
                        sel = sel + jnp.where(cum[l] <= iota, 1, 0)
                    sel = jnp.minimum(sel, 15)
                    pk_i = (s - RES).at[sel].get(mode="promise_in_bounds")
                    pk_p = pos.at[sel].get(mode="promise_in_bounds")
                    pk_i = jnp.where(iota < cntv, pk_i, pos & (NEWB - 1))
                    pk_p = jnp.where(iota < cntv, pk_p,
                                     PAD + (pos & (TRASH - 1)))
                    ibf[pl.ds(n_b, 16)] = pk_i
                    pbf[pl.ds(n_b, 16)] = pk_p
                n_b = n_b + cnt
            if g < 3:
                ga[g] = a_gather(g)
        for q in range(NBLK):
            for t in range(BBLK // 16):
                j = q * (BBLK // 16) + t
                ibg[q, pl.ds(t * 16, 16)] = ibf[pl.ds(j * 16, 16)]
                pbg[q, pl.ds(t * 16, 16)] = pbf[pl.ds(j * 16, 16)]
        gb0 = pltpu.async_copy(new_h.at[ibg.at[0]], buf_b, sem_b)
        wa = {}
        for g in range(SUBG):
            ga[g].wait()
            wa[g] = pltpu.async_copy(
                bufs_a[g % 3], out_h.at[pl.ds(base + g * BSUB, BSUB)], sem_w)
            if g >= 1 and g + 2 < SUBG:
                wa[g - 1].wait()
                ga[g + 2] = a_gather(g + 2)
        for g in range(max(0, SUBG - 3), SUBG):
            wa[g].wait()
        gb0.wait()

        @pl.when(n_b > 0)
        def _():
            pltpu.async_copy(buf_b, out_h.at[pbg.at[0]], sem_b).wait()
        for q in range(1, NBLK):
            @pl.when(n_b > q * BBLK)
            def _():
                pltpu.async_copy(new_h.at[ibg.at[q]], buf_b, sem_b).wait()
                pltpu.async_copy(buf_b, out_h.at[pbg.at[q]], sem_b).wait()

    return k(feat, new_feat, reserved, samp1d)


def kernel(feat, new_feat, reserved_ind, sampled_ind):
    pad = jnp.zeros((PAD - KEY,), dtype=sampled_ind.dtype)
    samp1d = jnp.concatenate([sampled_ind, pad])
    out = _sc_sample(feat, new_feat, reserved_ind, samp1d)
    return out[:KEY]


# speedup vs baseline: 1.1769x; 1.0162x over previous
import functools

import jax
import jax.numpy as jnp
from jax import lax
from jax.experimental import pallas as pl
from jax.experimental.pallas import tpu as pltpu
from jax.experimental.pallas import tpu_sc as plsc

MAXN = 200000
NEWB = 4096
RES = MAXN - NEWB
KEY = 20000

NC = 2
NS = 16
NW = NC * NS
BSUB = 128
SUBG = 5
CH = SUBG * BSUB
PAD = NW * CH
TRASH = 2048
GRP = BSUB // 16
BBLK = 64
NBLK = CH // BBLK


def _sc_sample(feat, new_feat, reserved, samp1d):
    mesh = plsc.VectorSubcoreMesh(core_axis_name="c", subcore_axis_name="s")

    @functools.partial(
        pl.kernel,
        mesh=mesh,
        out_type=jax.ShapeDtypeStruct((PAD + TRASH, 256), jnp.float32),
        scratch_types=[
            pltpu.VMEM((CH,), jnp.int32),
            pltpu.VMEM((SUBG, BSUB), jnp.int32),
            pltpu.VMEM((SUBG, BSUB), jnp.int32),
            pltpu.VMEM((SUBG, BSUB), jnp.int32),
            pltpu.VMEM((CH + 16,), jnp.int32),
            pltpu.VMEM((CH + 16,), jnp.int32),
            pltpu.VMEM((NBLK, BBLK), jnp.int32),
            pltpu.VMEM((NBLK, BBLK), jnp.int32),
            pltpu.VMEM((32,), jnp.int32),
            pltpu.VMEM((BSUB, 256), jnp.float32),
            pltpu.VMEM((BSUB, 256), jnp.float32),
            pltpu.VMEM((BSUB, 256), jnp.float32),
            pltpu.VMEM((BBLK, 256), jnp.float32),
            pltpu.SemaphoreType.DMA,
            pltpu.SemaphoreType.DMA,
            pltpu.SemaphoreType.DMA,
            pltpu.SemaphoreType.DMA,
        ],
    )
    def k(feat_h, new_h, res_h, samp_h, out_h,
          s_b, sc_b, r_b, ia_b, ibf, pbf, ibg, pbg, z_b,
          buf_a0, buf_a1, buf_a2, buf_b,
          sem_r, sem_ga, sem_w, sem_b):
        wid = lax.axis_index("s") * NC + lax.axis_index("c")
        base = wid * CH
        iota = lax.iota(jnp.int32, 16)
        pltpu.sync_copy(samp_h.at[pl.ds(wid * CH, CH)], s_b)
        z_b[pl.ds(0, 16)] = jnp.zeros((16,), jnp.int32)
        r_copies = []
        for g in range(SUBG):
            for t in range(GRP):
                j = g * GRP + t
                s = s_b[pl.ds(j * 16, 16)]
                pos = base + j * 16 + iota
                sc_b[g, pl.ds(t * 16, 16)] = jnp.minimum(s, RES - 1)
                ibf[pl.ds(j * 16, 16)] = pos & (NEWB - 1)
                pbf[pl.ds(j * 16, 16)] = PAD + (pos & (TRASH - 1))
            r_copies.append(
                pltpu.async_copy(res_h.at[sc_b.at[g]], r_b.at[g], sem_r))
        bufs_a = [buf_a0, buf_a1, buf_a2]

        def a_gather(g):
            return pltpu.async_copy(
                feat_h.at[ia_b.at[g]], bufs_a[g % 3], sem_ga)

        ga = {}
        n_b = jnp.int32(0)
        for g in range(SUBG):
            r_copies[g].wait()
            for t in range(GRP):
                j = g * GRP + t
                s = s_b[pl.ds(j * 16, 16)]
                r = r_b[g, pl.ds(t * 16, 16)]
                pos = base + j * 16 + iota
                m = s < RES
                ia_b[g, pl.ds(t * 16, 16)] = jnp.where(m, r, pos)
                mb = s >= RES
                cum = jnp.where(mb, 1, 0)
                for sh in (1, 2, 4, 8):
                    z_b[pl.ds(16, 16)] = cum
                    cum = cum + z_b[pl.ds(16 - sh, 16)]
                cnt = cum[15]

                @pl.when(cnt > 0)
                def _():
                    cntv = jnp.full((16,), cnt, jnp.int32)
                    sel = jnp.zeros((16,), jnp.int32)
                    for step in (8, 4, 2, 1):
                        cand = sel + step
                        c = cum.at[cand - 1].get(mode="promise_in_bounds")
                        sel = jnp.where(c <= iota, cand, sel)
                    sel = jnp.minimum(sel, 15)
                    pk_i = (s - RES).at[sel].get(mode="promise_in_bounds")
                    pk_p = pos.at[sel].get(mode="promise_in_bounds")
                    pk_i = jnp.where(iota < cntv, pk_i, pos & (NEWB - 1))
                    pk_p = jnp.where(iota < cntv, pk_p,
                                     PAD + (pos & (TRASH - 1)))
                    ibf[pl.ds(n_b, 16)] = pk_i
                    pbf[pl.ds(n_b, 16)] = pk_p
                n_b = n_b + cnt
            if g < 3:
                ga[g] = a_gather(g)
        for q in range(NBLK):
            for t in range(BBLK // 16):
                j = q * (BBLK // 16) + t
                ibg[q, pl.ds(t * 16, 16)] = ibf[pl.ds(j * 16, 16)]
                pbg[q, pl.ds(t * 16, 16)] = pbf[pl.ds(j * 16, 16)]
        gb0 = pltpu.async_copy(new_h.at[ibg.at[0]], buf_b, sem_b)
        wa = {}
        for g in range(SUBG):
            ga[g].wait()
            wa[g] = pltpu.async_copy(
                bufs_a[g % 3], out_h.at[pl.ds(base + g * BSUB, BSUB)], sem_w)
            if g >= 1 and g + 2 < SUBG:
                wa[g - 1].wait()
                ga[g + 2] = a_gather(g + 2)
        for g in range(max(0, SUBG - 3), SUBG):
            wa[g].wait()
        gb0.wait()

        @pl.when(n_b > 0)
        def _():
            pltpu.async_copy(buf_b, out_h.at[pbg.at[0]], sem_b).wait()
        for q in range(1, NBLK):
            @pl.when(n_b > q * BBLK)
            def _():
                pltpu.async_copy(new_h.at[ibg.at[q]], buf_b, sem_b).wait()
                pltpu.async_copy(buf_b, out_h.at[pbg.at[q]], sem_b).wait()

    return k(feat, new_feat, reserved, samp1d)


def kernel(feat, new_feat, reserved_ind, sampled_ind):
    pad = jnp.zeros((PAD - KEY,), dtype=sampled_ind.dtype)
    samp1d = jnp.concatenate([sampled_ind, pad])
    out = _sc_sample(feat, new_feat, reserved_ind, samp1d)
    return out[:KEY]
